# Initial kernel scaffold; baseline (speedup 1.0000x reference)
import jax
import jax.numpy as jnp
from jax.experimental import pallas as pl


def kernel(x, embed_weight, p_adic_scale):
    raise NotImplementedError("write your pallas kernel here")


# speedup vs baseline: 3.4995x; 3.4995x over previous
import functools

import jax
import jax.numpy as jnp
from jax import lax
from jax.experimental import pallas as pl
from jax.experimental.pallas import tpu as pltpu
from jax.experimental.pallas import tpu_sc as plsc

NC = 2
NS = 16
NW = NC * NS
LANES = 16

BATCH = 4096
HIST = 50
EMBED_DIM = 64
TOTAL = BATCH * HIST
PER_W = TOTAL // NW
CHUNK = 128
NCHUNK = PER_W // CHUNK
NBUF = 2


def _sc_body(table_hbm, idx_hbm, scale_hbm, out_hbm,
             idx_v, scale_v, buf0, buf1, sem0, sem1, idx_sem):
    wid = lax.axis_index("s") * NC + lax.axis_index("c")

    pltpu.async_copy(idx_hbm.at[wid], idx_v, idx_sem)
    pltpu.sync_copy(scale_hbm, scale_v)
    svecs = [scale_v[pl.ds(c * LANES, LANES)] for c in range(EMBED_DIM // LANES)]
    pltpu.make_async_copy(idx_hbm.at[wid], idx_v, idx_sem).wait()

    bufs = (buf0, buf1)
    sems = (sem0, sem1)

    def start(j, b):
        pltpu.async_copy(table_hbm.at[idx_v.at[j]], bufs[b], sems[b])

    def wait(b):
        pltpu.make_async_copy(table_hbm.at[idx_v.at[0]], bufs[b], sems[b]).wait()

    for b in range(NBUF):
        start(b, b)

    def scale_rows(b):
        buf = bufs[b]

        def row_body(r, carry):
            for c in range(EMBED_DIM // LANES):
                buf[r, pl.ds(c * LANES, LANES)] = (
                    buf[r, pl.ds(c * LANES, LANES)] * svecs[c]
                )
            return carry

        lax.fori_loop(0, CHUNK, row_body, 0, unroll=2)

    def superstep(s, carry):
        for b in range(NBUF):
            j = s * NBUF + b
            wait(b)
            scale_rows(b)
            pltpu.sync_copy(bufs[b], out_hbm.at[wid, j])

            @pl.when(j + NBUF < NCHUNK)
            def _():
                start(j + NBUF, b)

        return carry

    lax.fori_loop(0, NCHUNK // NBUF, superstep, 0)


@functools.partial(jax.jit, static_argnames=())
def _run(table, idx3, scale):
    mesh = plsc.VectorSubcoreMesh(
        core_axis_name="c", subcore_axis_name="s", num_cores=NC, num_subcores=NS
    )
    f = pl.kernel(
        _sc_body,
        out_type=jax.ShapeDtypeStruct((NW, NCHUNK, CHUNK, EMBED_DIM), jnp.float32),
        mesh=mesh,
        compiler_params=pltpu.CompilerParams(use_tc_tiling_on_sc=False),
        scratch_types=[
            pltpu.VMEM((NCHUNK, CHUNK), jnp.int32),
            pltpu.VMEM((EMBED_DIM,), jnp.float32),
            pltpu.VMEM((CHUNK, EMBED_DIM), jnp.float32),
            pltpu.VMEM((CHUNK, EMBED_DIM), jnp.float32),
            pltpu.SemaphoreType.DMA,
            pltpu.SemaphoreType.DMA,
            pltpu.SemaphoreType.DMA,
        ],
    )
    return f(table, idx3, scale)


def kernel(x, embed_weight, p_adic_scale):
    idx3 = x.astype(jnp.int32).reshape(NW, NCHUNK, CHUNK)
    out = _run(embed_weight, idx3, p_adic_scale)
    return out.reshape(BATCH, HIST, EMBED_DIM)
